# baseline (speedup 1.0000x reference)
import jax
import jax.numpy as jnp
from jax.experimental import pallas as pl


def _head_body(h2_ref, wc_ref, bc_ref, o_ref):
    o_ref[...] = jnp.tanh(h2_ref[...] @ wc_ref[...] + bc_ref[0, 0]) * 5.0


def kernel(x, edge_index, edge_attr, edge_type, Wn, bn, We, be, Wr1, Wer1, Wroot1, b1, Wr2, Wroot2, b2, Wc, bc):
    N = x.shape[0]
    src = edge_index[0]
    dst = edge_index[1]
    n = jax.nn.relu(x @ Wn + bn)
    e = jax.nn.relu(edge_attr @ We + be)
    nm = jnp.stack([n @ Wr1[0], n @ Wr1[1]], axis=0)
    node_msg = nm[edge_type, src]
    em0 = e @ Wer1[0]
    em1 = e @ Wer1[1]
    edge_msg = jnp.where((edge_type == 0)[:, None], em0, em1)
    msg = node_msg + edge_msg
    agg = jax.ops.segment_sum(msg, dst, num_segments=N)
    deg = jax.ops.segment_sum(jnp.ones((msg.shape[0],), jnp.float32), dst, num_segments=N)
    agg = agg / jnp.maximum(deg, 1.0)[:, None]
    h = jax.nn.relu(agg + n @ Wroot1 + b1)
    nm2 = jnp.stack([h @ Wr2[0], h @ Wr2[1]], axis=0)
    msg2 = nm2[edge_type, src]
    agg2 = jax.ops.segment_max(msg2, dst, num_segments=N)
    agg2 = jnp.where(jnp.isfinite(agg2), agg2, 0.0)
    h2 = jax.nn.relu(agg2 + h @ Wroot2 + b2)
    out = pl.pallas_call(
        _head_body,
        out_shape=jax.ShapeDtypeStruct((N, 1), jnp.float32),
        grid=(N // 10000,),
        in_specs=[
            pl.BlockSpec((10000, 64), lambda i: (i, 0)),
            pl.BlockSpec((64, 1), lambda i: (0, 0)),
            pl.BlockSpec((1, 1), lambda i: (0, 0)),
        ],
        out_specs=pl.BlockSpec((10000, 1), lambda i: (i, 0)),
    )(h2, Wc, bc.reshape(1, 1))
    return out


# speedup vs baseline: 4.0101x; 4.0101x over previous
import functools

import jax
import jax.numpy as jnp
from jax import lax
from jax.experimental import pallas as pl
from jax.experimental.pallas import tpu as pltpu
from jax.experimental.pallas import tpu_sc as plsc

NS = 100352
RB = 512
NBUCK = 196
NBP = 224
CH = 512
EB = 12800
NGRID = 125
TRASH = 512
MP = 1663488
MPOS0 = 1625088
MP2 = 1638400
ACC1W = 160
NEG = -3.0e38


def _hr_body(dst_ref, hist_ref, rank_ref):
    d = dst_ref[0, 0, :]
    b = lax.shift_right_logical(d, 9).reshape(25, CH)
    iota_b = lax.broadcasted_iota(jnp.int32, (1, NBP), 1)
    lt = lax.broadcasted_iota(jnp.int32, (CH, CH), 0) > lax.broadcasted_iota(
        jnp.int32, (CH, CH), 1)
    L = lt.astype(jnp.bfloat16)
    ranks = []
    hists = []
    for c in range(25):
        M = (b[c][:, None] == iota_b).astype(jnp.float32)
        hists.append(jnp.sum(M, axis=0))
        C = lax.dot_general(L, M.astype(jnp.bfloat16), (((1,), (0,)), ((), ())),
                            preferred_element_type=jnp.float32)
        ranks.append(jnp.sum(M * C, axis=1))
    hist_ref[0] = jnp.stack(hists)
    rank_ref[0, 0] = jnp.concatenate(ranks)


def _scan_body(hist_ref, co_ref, tot_ref, carry_ref):
    g = pl.program_id(0)

    @pl.when(g == 0)
    def _():
        carry_ref[...] = jnp.zeros((8, NBP), jnp.float32)

    h = hist_ref[...]
    lt128 = (lax.broadcasted_iota(jnp.int32, (128, 128), 0)
             > lax.broadcasted_iota(jnp.int32, (128, 128), 1))
    L128 = lt128.astype(jnp.bfloat16)
    within = lax.dot_general(L128, h.astype(jnp.bfloat16),
                             (((1,), (0,)), ((), ())),
                             preferred_element_type=jnp.float32)
    carry = carry_ref[...]
    co_ref[...] = within + carry[0:1, :]
    s = jnp.sum(h, axis=0, keepdims=True)
    carry2 = carry + jnp.broadcast_to(s, (8, NBP))
    carry_ref[...] = carry2
    tot_ref[...] = carry2[0:1, :]


def _base_body(tot_ref, starts_ref, cnts_ref):
    tot = tot_ref[0, :]
    sz = jnp.ceil(tot * (1.0 / 128.0)) * 128.0
    ltB = (lax.broadcasted_iota(jnp.int32, (NBP, NBP), 0)
           > lax.broadcasted_iota(jnp.int32, (NBP, NBP), 1)).astype(jnp.float32)
    base = jnp.sum(ltB * sz[None, :], axis=1)
    sp = jnp.concatenate([base, jnp.full((32,), float(MPOS0), jnp.float32)])
    starts_ref[...] = sp.astype(jnp.int32).reshape(1, 256)
    cnts_ref[...] = jnp.concatenate(
        [tot, jnp.zeros((32,), jnp.float32)]).astype(jnp.int32).reshape(1, 256)


def _pos_body(dst_ref, src_ref, typ_ref, rank_ref, co_ref, starts_ref,
              pos_ref, meta_ref):
    d = dst_ref[0, 0, :]
    b = lax.shift_right_logical(d, 9).reshape(25, CH)
    dl = jnp.bitwise_and(d, RB - 1)
    iota_b = lax.broadcasted_iota(jnp.int32, (1, NBP), 1)
    basef = starts_ref[0, 0:NBP].astype(jnp.float32)
    offs = []
    for c in range(25):
        M = (b[c][:, None] == iota_b).astype(jnp.float32)
        offs.append(jnp.sum(M * (co_ref[0, c] + basef)[None, :], axis=1))
    pos = jnp.concatenate(offs) + rank_ref[0, 0]
    pos_ref[0, 0] = pos.astype(jnp.int32)
    meta_ref[0, 0] = (src_ref[0, 0, :]
                      + lax.shift_left(typ_ref[0, 0, :], 17)
                      + lax.shift_left(dl, 18))


def _tc1_body(x_ref, wn_ref, bn_ref, wr1_ref, n_ref, nm_ref):
    nv = jax.nn.relu(
        jnp.dot(x_ref[...], wn_ref[...], preferred_element_type=jnp.float32, precision=lax.Precision.HIGHEST)
        + bn_ref[...])
    n_ref[...] = nv
    nm_ref[...] = jnp.concatenate(
        [jnp.dot(nv, wr1_ref[0], preferred_element_type=jnp.float32, precision=lax.Precision.HIGHEST),
         jnp.dot(nv, wr1_ref[1], preferred_element_type=jnp.float32, precision=lax.Precision.HIGHEST)], axis=1)


def _tc3_body(agg2_ref, h_ref, wroot2_ref, b2_ref, wc_ref, bc_ref, o_ref):
    a2 = agg2_ref[...]
    a2 = jnp.where(a2 > NEG, a2, 0.0)
    h2 = jax.nn.relu(
        a2 + jnp.dot(h_ref[...], wroot2_ref[...],
                     preferred_element_type=jnp.float32,
                     precision=lax.Precision.HIGHEST) + b2_ref[...])
    o_ref[...] = jnp.tanh(
        jnp.dot(h2, wc_ref[...], preferred_element_type=jnp.float32,
                precision=lax.Precision.HIGHEST)
        + bc_ref[...]) * 5.0


def _mesh():
    return plsc.VectorSubcoreMesh(core_axis_name="c", subcore_axis_name="s")


def _wid():
    return lax.axis_index("s") * 2 + lax.axis_index("c")


def _scatter_body(pos_hbm, meta_hbm, a0_hbm, a1_hbm,
                  metaP_hbm, a0P_hbm, a1P_hbm,
                  posv, mv, a0v, a1v, sem, sem2, sem3):
    wid = _wid()
    base = wid * 25 * 16
    def chunk(k, _):
        row = base + k * 16
        pltpu.sync_copy(pos_hbm.at[pl.ds(row, 16)], posv)
        pltpu.sync_copy(meta_hbm.at[pl.ds(row, 16)], mv)
        pltpu.sync_copy(a0_hbm.at[pl.ds(row, 16)], a0v)
        pltpu.sync_copy(a1_hbm.at[pl.ds(row, 16)], a1v)
        cps = []
        for j in range(16):
            cps.append(pltpu.async_copy(
                mv.at[j], metaP_hbm.at[posv.at[j]], sem))
            cps.append(pltpu.async_copy(
                a0v.at[j], a0P_hbm.at[posv.at[j]], sem2))
            cps.append(pltpu.async_copy(
                a1v.at[j], a1P_hbm.at[posv.at[j]], sem3))
        for cp in cps:
            cp.wait()
        return 0
    lax.fori_loop(0, 25, chunk, 0)


def _conv1_body(meta_hbm, a0_hbm, a1_hbm, nmp_hbm, wpack_hbm, st_hbm,
                zero_hbm,
                out_hbm,
                accv, metav, a0v, a1v, idxv, rowsv, wv, stv, sem):
    wid = _wid()
    pltpu.sync_copy(wpack_hbm, wv)
    pltpu.sync_copy(st_hbm, stv.at[pl.ds(0, 512)])
    w00 = wv[pl.ds(0, 16)]
    w01 = wv[pl.ds(16, 16)]
    w10 = wv[pl.ds(32, 16)]
    w11 = wv[pl.ds(48, 16)]
    wb0 = wv[pl.ds(64, 16)]
    wb1 = wv[pl.ds(80, 16)]
    cunit = wv[pl.ds(96, 16)]

    def bucket(kb, _):
        b = wid + kb * 32

        @pl.when(b < NBUCK)
        def _():
            sv = stv[pl.ds(b * 2, 16)]
            start = pl.multiple_of(sv[0], 128)
            cnt = sv[1]
            pltpu.sync_copy(zero_hbm, accv)
            nch = lax.div(cnt + 127, 128)

            def chunk(kc, _):
                st = pl.multiple_of(start + kc * 128, 128)
                pltpu.sync_copy(meta_hbm.at[pl.ds(st, 128)],
                                metav.at[pl.ds(0, 128)])
                pltpu.sync_copy(a0_hbm.at[pl.ds(st, 128)],
                                a0v.at[pl.ds(0, 128)])
                pltpu.sync_copy(a1_hbm.at[pl.ds(st, 128)],
                                a1v.at[pl.ds(0, 128)])

                def ib(j, _):
                    m = metav[pl.ds(j * 16, 16)]
                    srcv = jnp.bitwise_and(m, 0x1FFFF)
                    idxv[pl.ds(j * 16, 16)] = jnp.minimum(srcv, NS - 1)
                    return 0
                lax.fori_loop(0, 8, ib, 0, unroll=True)
                pltpu.async_copy(nmp_hbm.at[idxv], rowsv, sem).wait()
                rem = cnt - kc * 128

                def edge(i, _):
                    m = metav[pl.ds(i, 16)][0]
                    dl = lax.shift_right_logical(m, 18)
                    t = jnp.bitwise_and(lax.shift_right_logical(m, 17), 1)
                    dlv = jnp.where(i < rem, jnp.minimum(dl, TRASH), TRASH)
                    base = dlv * ACC1W
                    for c in range(4):
                        v = rowsv[i, pl.ds(t * 64 + c * 16, 16)]
                        plsc.addupdate(accv.at[pl.ds(base + c * 16, 16)], v)
                    a0 = a0v[pl.ds(i, 16)][0]
                    a1 = a1v[pl.ds(i, 16)][0]
                    eb = base + 64 + t * 48
                    ef0 = jnp.maximum(a0 * w00 + a1 * w10 + wb0, 0.0)
                    plsc.addupdate(accv.at[pl.ds(eb, 16)], ef0)
                    ef1 = jnp.maximum(a0 * w01 + a1 * w11 + wb1, 0.0)
                    plsc.addupdate(accv.at[pl.ds(eb + 16, 16)], ef1)
                    plsc.addupdate(accv.at[pl.ds(eb + 32, 16)], cunit)
                    return 0
                lax.fori_loop(0, 128, edge, 0)
                return 0
            lax.fori_loop(0, nch, chunk, 0)
            off = pl.multiple_of(b * (RB * ACC1W), 128)
            pltpu.sync_copy(accv.at[pl.ds(0, RB * ACC1W)],
                            out_hbm.at[pl.ds(off, RB * ACC1W)])
        return 0
    lax.fori_loop(0, 7, bucket, 0)


def _conv2_body(meta_hbm, nm2_hbm, st_hbm, neg_hbm,
                out_hbm,
                accv, metav, idxv, rowsv, stv, sem):
    wid = _wid()
    pltpu.sync_copy(st_hbm, stv.at[pl.ds(0, 512)])

    def bucket(kb, _):
        b = wid + kb * 32

        @pl.when(b < NBUCK)
        def _():
            sv = stv[pl.ds(b * 2, 16)]
            start = pl.multiple_of(sv[0], 128)
            cnt = sv[1]
            pltpu.sync_copy(neg_hbm, accv)
            nch = lax.div(cnt + 127, 128)

            def chunk(kc, _):
                st = pl.multiple_of(start + kc * 128, 128)
                pltpu.sync_copy(meta_hbm.at[pl.ds(st, 128)],
                                metav.at[pl.ds(0, 128)])

                def ib(j, _):
                    m = metav[pl.ds(j * 16, 16)]
                    srcv = jnp.bitwise_and(m, 0x1FFFF)
                    idxv[pl.ds(j * 16, 16)] = jnp.minimum(srcv, NS - 1)
                    return 0
                lax.fori_loop(0, 8, ib, 0, unroll=True)
                pltpu.async_copy(nm2_hbm.at[idxv], rowsv, sem).wait()
                rem = cnt - kc * 128

                def edge(i, _):
                    m = metav[pl.ds(i, 16)][0]
                    dl = lax.shift_right_logical(m, 18)
                    t = jnp.bitwise_and(lax.shift_right_logical(m, 17), 1)
                    dlv = jnp.where(i < rem, jnp.minimum(dl, TRASH), TRASH)
                    base = dlv * 64
                    for c in range(4):
                        v = rowsv[i, pl.ds(t * 64 + c * 16, 16)]
                        a = accv[pl.ds(base + c * 16, 16)]
                        accv[pl.ds(base + c * 16, 16)] = jnp.maximum(a, v)
                    return 0
                lax.fori_loop(0, 128, edge, 0)
                return 0
            lax.fori_loop(0, nch, chunk, 0)
            off = pl.multiple_of(b * (RB * 64), 128)
            pltpu.sync_copy(accv.at[pl.ds(0, RB * 64)],
                            out_hbm.at[pl.ds(off, RB * 64)])
        return 0
    lax.fori_loop(0, 7, bucket, 0)


def kernel(x, edge_index, edge_attr, edge_type, Wn, bn, We, be, Wr1, Wer1,
           Wroot1, b1, Wr2, Wroot2, b2, Wc, bc):
    N = x.shape[0]
    E = edge_index.shape[1]
    f32 = jnp.float32

    src = edge_index[0]
    dst = edge_index[1]
    dst3 = dst.reshape(NGRID, 1, EB)
    src3 = src.reshape(NGRID, 1, EB)
    typ3 = edge_type.reshape(NGRID, 1, EB)

    hist3, rank3 = pl.pallas_call(
        _hr_body,
        grid=(NGRID,),
        in_specs=[pl.BlockSpec((1, 1, EB), lambda i: (i, 0, 0))],
        out_specs=[pl.BlockSpec((1, 25, NBP), lambda i: (i, 0, 0)),
                   pl.BlockSpec((1, 1, EB), lambda i: (i, 0, 0))],
        out_shape=[jax.ShapeDtypeStruct((NGRID, 25, NBP), f32),
                   jax.ShapeDtypeStruct((NGRID, 1, EB), f32)],
    )(dst3)

    hist = jnp.pad(hist3.reshape(3125, NBP), ((0, 75), (0, 0)))
    co, totI = pl.pallas_call(
        _scan_body,
        grid=(25,),
        in_specs=[pl.BlockSpec((128, NBP), lambda g: (g, 0))],
        out_specs=[pl.BlockSpec((128, NBP), lambda g: (g, 0)),
                   pl.BlockSpec((1, NBP), lambda g: (0, 0))],
        out_shape=[jax.ShapeDtypeStruct((3200, NBP), f32),
                   jax.ShapeDtypeStruct((1, NBP), f32)],
        scratch_shapes=[pltpu.VMEM((8, NBP), f32)],
    )(hist)
    startsI, cntsI = pl.pallas_call(
        _base_body,
        in_specs=[pl.BlockSpec((1, NBP), lambda: (0, 0))],
        out_specs=[pl.BlockSpec((1, 256), lambda: (0, 0)),
                   pl.BlockSpec((1, 256), lambda: (0, 0))],
        out_shape=[jax.ShapeDtypeStruct((1, 256), jnp.int32),
                   jax.ShapeDtypeStruct((1, 256), jnp.int32)],
    )(totI)
    co3 = co[:3125].reshape(NGRID, 25, NBP)

    pos3, meta3 = pl.pallas_call(
        _pos_body,
        grid=(NGRID,),
        in_specs=[pl.BlockSpec((1, 1, EB), lambda i: (i, 0, 0)),
                  pl.BlockSpec((1, 1, EB), lambda i: (i, 0, 0)),
                  pl.BlockSpec((1, 1, EB), lambda i: (i, 0, 0)),
                  pl.BlockSpec((1, 1, EB), lambda i: (i, 0, 0)),
                  pl.BlockSpec((1, 25, NBP), lambda i: (i, 0, 0)),
                  pl.BlockSpec((1, 256), lambda i: (0, 0))],
        out_specs=[pl.BlockSpec((1, 1, EB), lambda i: (i, 0, 0)),
                   pl.BlockSpec((1, 1, EB), lambda i: (i, 0, 0))],
        out_shape=[jax.ShapeDtypeStruct((NGRID, 1, EB), jnp.int32),
                   jax.ShapeDtypeStruct((NGRID, 1, EB), jnp.int32)],
    )(dst3, src3, typ3, rank3, co3, startsI)

    npad = MP2 - E
    pos_pad = jnp.concatenate(
        [pos3.reshape(E), MPOS0 + jnp.arange(npad, dtype=jnp.int32)])
    meta_pad = jnp.concatenate(
        [meta3.reshape(E),
         jnp.full((npad,), TRASH << 18, jnp.int32)])
    a0_pad = jnp.concatenate([edge_attr[:, 0], jnp.zeros((npad,), f32)])
    a1_pad = jnp.concatenate([edge_attr[:, 1], jnp.zeros((npad,), f32)])

    xp = jnp.pad(x, ((0, NS - N), (0, 8 - x.shape[1])))
    Wnp = jnp.pad(Wn, ((0, 8 - Wn.shape[0]), (0, 0)))
    n, nmp = pl.pallas_call(
        _tc1_body,
        grid=(98,),
        in_specs=[pl.BlockSpec((1024, 8), lambda i: (i, 0)),
                  pl.BlockSpec((8, 64), lambda i: (0, 0)),
                  pl.BlockSpec((1, 64), lambda i: (0, 0)),
                  pl.BlockSpec((2, 64, 64), lambda i: (0, 0, 0))],
        out_specs=[pl.BlockSpec((1024, 64), lambda i: (i, 0)),
                   pl.BlockSpec((1024, 128), lambda i: (i, 0))],
        out_shape=[jax.ShapeDtypeStruct((NS, 64), f32),
                   jax.ShapeDtypeStruct((NS, 128), f32)],
    )(xp, Wnp, bn.reshape(1, 64), Wr1)

    metaP, a0P, a1P = pl.kernel(
        _scatter_body,
        out_type=[jax.ShapeDtypeStruct((MP,), jnp.int32),
                  jax.ShapeDtypeStruct((MP,), f32),
                  jax.ShapeDtypeStruct((MP,), f32)],
        mesh=_mesh(),
        scratch_types=[pltpu.VMEM((16, 128), jnp.int32),
                       pltpu.VMEM((16, 128), jnp.int32),
                       pltpu.VMEM((16, 128), f32),
                       pltpu.VMEM((16, 128), f32),
                       pltpu.SemaphoreType.DMA,
                       pltpu.SemaphoreType.DMA,
                       pltpu.SemaphoreType.DMA],
    )(pos_pad.reshape(12800, 128), meta_pad.reshape(12800, 128),
      a0_pad.reshape(12800, 128), a1_pad.reshape(12800, 128))

    wpack = jnp.concatenate(
        [We[0], We[1], be,
         jnp.zeros((32,), f32).at[0].set(1.0)]).reshape(128)
    stpack = jnp.concatenate(
        [startsI.reshape(256, 1), cntsI.reshape(256, 1)], axis=1).reshape(512)
    acc1 = pl.kernel(
        _conv1_body,
        out_type=jax.ShapeDtypeStruct((NS * ACC1W,), f32),
        mesh=_mesh(),
        scratch_types=[pltpu.VMEM(((RB + 1) * ACC1W,), f32),
                       pltpu.VMEM((144,), jnp.int32),
                       pltpu.VMEM((144,), f32),
                       pltpu.VMEM((144,), f32),
                       pltpu.VMEM((128,), jnp.int32),
                       pltpu.VMEM((128, 128), f32),
                       pltpu.VMEM((128,), f32),
                       pltpu.VMEM((528,), jnp.int32),
                       pltpu.SemaphoreType.DMA],
    )(metaP, a0P, a1P, nmp, wpack, stpack,
      jnp.zeros(((RB + 1) * ACC1W,), f32))

    h, nm2p = pl.pallas_call(
        _tc2_body_real,
        grid=(98,),
        in_specs=[pl.BlockSpec((1024, ACC1W), lambda i: (i, 0)),
                  pl.BlockSpec((1024, 64), lambda i: (i, 0)),
                  pl.BlockSpec((2, 32, 64), lambda i: (0, 0, 0)),
                  pl.BlockSpec((64, 64), lambda i: (0, 0)),
                  pl.BlockSpec((1, 64), lambda i: (0, 0)),
                  pl.BlockSpec((2, 64, 64), lambda i: (0, 0, 0))],
        out_specs=[pl.BlockSpec((1024, 64), lambda i: (i, 0)),
                   pl.BlockSpec((1024, 128), lambda i: (i, 0))],
        out_shape=[jax.ShapeDtypeStruct((NS, 64), f32),
                   jax.ShapeDtypeStruct((NS, 128), f32)],
    )(acc1.reshape(NS, ACC1W), n, Wer1, Wroot1, b1.reshape(1, 64), Wr2)

    agg2f = pl.kernel(
        _conv2_body,
        out_type=jax.ShapeDtypeStruct((NS * 64,), f32),
        mesh=_mesh(),
        scratch_types=[pltpu.VMEM(((RB + 1) * 64,), f32),
                       pltpu.VMEM((144,), jnp.int32),
                       pltpu.VMEM((128,), jnp.int32),
                       pltpu.VMEM((128, 128), f32),
                       pltpu.VMEM((528,), jnp.int32),
                       pltpu.SemaphoreType.DMA],
    )(metaP, nm2p, stpack, jnp.full(((RB + 1) * 64,), NEG, f32))

    Wc128 = jnp.pad(Wc, ((0, 0), (0, 127)))
    bc128 = jnp.pad(bc.reshape(1, 1), ((0, 0), (0, 127)))
    out = pl.pallas_call(
        _tc3_body,
        grid=(98,),
        in_specs=[pl.BlockSpec((1024, 64), lambda i: (i, 0)),
                  pl.BlockSpec((1024, 64), lambda i: (i, 0)),
                  pl.BlockSpec((64, 64), lambda i: (0, 0)),
                  pl.BlockSpec((1, 64), lambda i: (0, 0)),
                  pl.BlockSpec((64, 128), lambda i: (0, 0)),
                  pl.BlockSpec((1, 128), lambda i: (0, 0))],
        out_specs=pl.BlockSpec((1024, 128), lambda i: (i, 0)),
        out_shape=jax.ShapeDtypeStruct((NS, 128), f32),
    )(agg2f.reshape(NS, 64), h, Wroot2, b2.reshape(1, 64), Wc128, bc128)
    return out[:N, :1]


def _tc2_body_real(acc_ref, n_ref, wer1_ref, wroot1_ref, b1_ref, wr2_ref,
                   h_ref, nm2_ref):
    acc = acc_ref[...]
    nm_s = acc[:, 0:64]
    e0 = acc[:, 64:96]
    c0 = acc[:, 96:97]
    e1 = acc[:, 112:144]
    c1 = acc[:, 144:145]
    deg = jnp.maximum(c0 + c1, 1.0)
    agg = (nm_s
           + jnp.dot(e0, wer1_ref[0], preferred_element_type=jnp.float32, precision=lax.Precision.HIGHEST)
           + jnp.dot(e1, wer1_ref[1], preferred_element_type=jnp.float32, precision=lax.Precision.HIGHEST)
           ) / deg
    hv = jax.nn.relu(
        agg + jnp.dot(n_ref[...], wroot1_ref[...],
                      preferred_element_type=jnp.float32,
                      precision=lax.Precision.HIGHEST) + b1_ref[...])
    h_ref[...] = hv
    nm2_ref[...] = jnp.concatenate(
        [jnp.dot(hv, wr2_ref[0], preferred_element_type=jnp.float32, precision=lax.Precision.HIGHEST),
         jnp.dot(hv, wr2_ref[1], preferred_element_type=jnp.float32, precision=lax.Precision.HIGHEST)], axis=1)


# speedup vs baseline: 4.0171x; 1.0017x over previous
import functools

import jax
import jax.numpy as jnp
from jax import lax
from jax.experimental import pallas as pl
from jax.experimental.pallas import tpu as pltpu
from jax.experimental.pallas import tpu_sc as plsc

NS = 100352
RB = 512
NBUCK = 196
NBP = 224
CH = 512
EB = 12800
NGRID = 125
TRASH = 512
MP = 1663488
MPOS0 = 1625088
MP2 = 1638400
ACC1W = 160
NEG = -3.0e38


def _hr_body(dst_ref, hist_ref, rank_ref):
    d = dst_ref[0, 0, :]
    b = lax.shift_right_logical(d, 9).reshape(25, CH)
    iota_b = lax.broadcasted_iota(jnp.int32, (1, NBP), 1)
    lt = lax.broadcasted_iota(jnp.int32, (CH, CH), 0) > lax.broadcasted_iota(
        jnp.int32, (CH, CH), 1)
    L = lt.astype(jnp.bfloat16)
    ranks = []
    hists = []
    for c in range(25):
        M = (b[c][:, None] == iota_b).astype(jnp.float32)
        hists.append(jnp.sum(M, axis=0))
        C = lax.dot_general(L, M.astype(jnp.bfloat16), (((1,), (0,)), ((), ())),
                            preferred_element_type=jnp.float32)
        ranks.append(jnp.sum(M * C, axis=1))
    hist_ref[0] = jnp.stack(hists)
    rank_ref[0, 0] = jnp.concatenate(ranks)


def _scan_body(hist_ref, co_ref, tot_ref, carry_ref):
    g = pl.program_id(0)

    @pl.when(g == 0)
    def _():
        carry_ref[...] = jnp.zeros((8, NBP), jnp.float32)

    h = hist_ref[...]
    lt128 = (lax.broadcasted_iota(jnp.int32, (128, 128), 0)
             > lax.broadcasted_iota(jnp.int32, (128, 128), 1))
    L128 = lt128.astype(jnp.bfloat16)
    within = lax.dot_general(L128, h.astype(jnp.bfloat16),
                             (((1,), (0,)), ((), ())),
                             preferred_element_type=jnp.float32)
    carry = carry_ref[...]
    co_ref[...] = within + carry[0:1, :]
    s = jnp.sum(h, axis=0, keepdims=True)
    carry2 = carry + jnp.broadcast_to(s, (8, NBP))
    carry_ref[...] = carry2
    tot_ref[...] = carry2[0:1, :]


def _base_body(tot_ref, starts_ref, cnts_ref):
    tot = tot_ref[0, :]
    sz = jnp.ceil(tot * (1.0 / 128.0)) * 128.0
    ltB = (lax.broadcasted_iota(jnp.int32, (NBP, NBP), 0)
           > lax.broadcasted_iota(jnp.int32, (NBP, NBP), 1)).astype(jnp.float32)
    base = jnp.sum(ltB * sz[None, :], axis=1)
    sp = jnp.concatenate([base, jnp.full((32,), float(MPOS0), jnp.float32)])
    starts_ref[...] = sp.astype(jnp.int32).reshape(1, 256)
    cnts_ref[...] = jnp.concatenate(
        [tot, jnp.zeros((32,), jnp.float32)]).astype(jnp.int32).reshape(1, 256)


def _pos_body(dst_ref, src_ref, typ_ref, rank_ref, co_ref, starts_ref,
              pos_ref, meta_ref):
    d = dst_ref[0, 0, :]
    b = lax.shift_right_logical(d, 9).reshape(25, CH)
    dl = jnp.bitwise_and(d, RB - 1)
    iota_b = lax.broadcasted_iota(jnp.int32, (1, NBP), 1)
    basef = starts_ref[0, 0:NBP].astype(jnp.float32)
    offs = []
    for c in range(25):
        M = (b[c][:, None] == iota_b).astype(jnp.float32)
        offs.append(jnp.sum(M * (co_ref[0, c] + basef)[None, :], axis=1))
    pos = jnp.concatenate(offs) + rank_ref[0, 0]
    pos_ref[0, 0] = pos.astype(jnp.int32)
    meta_ref[0, 0] = (src_ref[0, 0, :]
                      + lax.shift_left(typ_ref[0, 0, :], 17)
                      + lax.shift_left(dl, 18))


def _tc1_body(x_ref, wn_ref, bn_ref, wr1_ref, n_ref, nm_ref):
    nv = jax.nn.relu(
        jnp.dot(x_ref[...], wn_ref[...], preferred_element_type=jnp.float32, precision=lax.Precision.HIGHEST)
        + bn_ref[...])
    n_ref[...] = nv
    nm_ref[...] = jnp.concatenate(
        [jnp.dot(nv, wr1_ref[0], preferred_element_type=jnp.float32, precision=lax.Precision.HIGHEST),
         jnp.dot(nv, wr1_ref[1], preferred_element_type=jnp.float32, precision=lax.Precision.HIGHEST)], axis=1)


def _tc3_body(agg2_ref, h_ref, wroot2_ref, b2_ref, wc_ref, bc_ref, o_ref):
    a2 = agg2_ref[...]
    a2 = jnp.where(a2 > NEG, a2, 0.0)
    h2 = jax.nn.relu(
        a2 + jnp.dot(h_ref[...], wroot2_ref[...],
                     preferred_element_type=jnp.float32,
                     precision=lax.Precision.HIGHEST) + b2_ref[...])
    o_ref[...] = jnp.tanh(
        jnp.dot(h2, wc_ref[...], preferred_element_type=jnp.float32,
                precision=lax.Precision.HIGHEST)
        + bc_ref[...]) * 5.0


def _mesh():
    return plsc.VectorSubcoreMesh(core_axis_name="c", subcore_axis_name="s")


def _wid():
    return lax.axis_index("s") * 2 + lax.axis_index("c")


def _scatter_body(pos_hbm, meta_hbm, a0_hbm, a1_hbm,
                  metaP_hbm, a0P_hbm, a1P_hbm,
                  posv, mv, a0v, a1v, sem, sem2, sem3):
    wid = _wid()
    base = wid * 25 * 16
    def chunk(k, _):
        row = base + k * 16
        pltpu.sync_copy(pos_hbm.at[pl.ds(row, 16)], posv)
        pltpu.sync_copy(meta_hbm.at[pl.ds(row, 16)], mv)
        pltpu.sync_copy(a0_hbm.at[pl.ds(row, 16)], a0v)
        pltpu.sync_copy(a1_hbm.at[pl.ds(row, 16)], a1v)
        cps = []
        for j in range(16):
            cps.append(pltpu.async_copy(
                mv.at[j], metaP_hbm.at[posv.at[j]], sem))
            cps.append(pltpu.async_copy(
                a0v.at[j], a0P_hbm.at[posv.at[j]], sem2))
            cps.append(pltpu.async_copy(
                a1v.at[j], a1P_hbm.at[posv.at[j]], sem3))
        for cp in cps:
            cp.wait()
        return 0
    lax.fori_loop(0, 25, chunk, 0)


def _conv1_body(meta_hbm, a0_hbm, a1_hbm, nmp_hbm, wpack_hbm, st_hbm,
                zero_hbm,
                out_hbm,
                accv, metav, a0v, a1v, idxv, rowsv, wv, stv, sem):
    wid = _wid()
    pltpu.sync_copy(wpack_hbm, wv)
    pltpu.sync_copy(st_hbm, stv.at[pl.ds(0, 512)])
    w00 = wv[pl.ds(0, 16)]
    w01 = wv[pl.ds(16, 16)]
    w10 = wv[pl.ds(32, 16)]
    w11 = wv[pl.ds(48, 16)]
    wb0 = wv[pl.ds(64, 16)]
    wb1 = wv[pl.ds(80, 16)]
    cunit = wv[pl.ds(96, 16)]

    def bucket(kb, _):
        b = wid + kb * 32

        @pl.when(b < NBUCK)
        def _():
            sv = stv[pl.ds(b * 2, 16)]
            start = pl.multiple_of(sv[0], 128)
            cnt = sv[1]
            pltpu.sync_copy(zero_hbm, accv)
            nch = lax.div(cnt + 127, 128)

            def chunk(kc, _):
                st = pl.multiple_of(start + kc * 128, 128)
                pltpu.sync_copy(meta_hbm.at[pl.ds(st, 128)],
                                metav.at[pl.ds(0, 128)])
                pltpu.sync_copy(a0_hbm.at[pl.ds(st, 128)],
                                a0v.at[pl.ds(0, 128)])
                pltpu.sync_copy(a1_hbm.at[pl.ds(st, 128)],
                                a1v.at[pl.ds(0, 128)])

                def ib(j, _):
                    m = metav[pl.ds(j * 16, 16)]
                    srcv = jnp.bitwise_and(m, 0x1FFFF)
                    idxv[pl.ds(j * 16, 16)] = jnp.minimum(srcv, NS - 1)
                    return 0
                lax.fori_loop(0, 8, ib, 0, unroll=True)
                pltpu.async_copy(nmp_hbm.at[idxv], rowsv, sem).wait()
                rem = cnt - kc * 128

                def edge(i, _):
                    m = metav[pl.ds(i, 16)][0]
                    dl = lax.shift_right_logical(m, 18)
                    t = jnp.bitwise_and(lax.shift_right_logical(m, 17), 1)
                    dlv = jnp.where(i < rem, jnp.minimum(dl, TRASH), TRASH)
                    base = dlv * ACC1W
                    for c in range(4):
                        v = rowsv[i, pl.ds(t * 64 + c * 16, 16)]
                        plsc.addupdate(accv.at[pl.ds(base + c * 16, 16)], v)
                    a0 = a0v[pl.ds(i, 16)][0]
                    a1 = a1v[pl.ds(i, 16)][0]
                    eb = base + 64 + t * 48
                    ef0 = jnp.maximum(a0 * w00 + a1 * w10 + wb0, 0.0)
                    plsc.addupdate(accv.at[pl.ds(eb, 16)], ef0)
                    ef1 = jnp.maximum(a0 * w01 + a1 * w11 + wb1, 0.0)
                    plsc.addupdate(accv.at[pl.ds(eb + 16, 16)], ef1)
                    plsc.addupdate(accv.at[pl.ds(eb + 32, 16)], cunit)
                    return 0
                lax.fori_loop(0, 128, edge, 0, unroll=4)
                return 0
            lax.fori_loop(0, nch, chunk, 0)
            off = pl.multiple_of(b * (RB * ACC1W), 128)
            pltpu.sync_copy(accv.at[pl.ds(0, RB * ACC1W)],
                            out_hbm.at[pl.ds(off, RB * ACC1W)])
        return 0
    lax.fori_loop(0, 7, bucket, 0)


def _conv2_body(meta_hbm, nm2_hbm, st_hbm, neg_hbm,
                out_hbm,
                accv, metav, idxv, rowsv, stv, sem):
    wid = _wid()
    pltpu.sync_copy(st_hbm, stv.at[pl.ds(0, 512)])

    def bucket(kb, _):
        b = wid + kb * 32

        @pl.when(b < NBUCK)
        def _():
            sv = stv[pl.ds(b * 2, 16)]
            start = pl.multiple_of(sv[0], 128)
            cnt = sv[1]
            pltpu.sync_copy(neg_hbm, accv)
            nch = lax.div(cnt + 127, 128)

            def chunk(kc, _):
                st = pl.multiple_of(start + kc * 128, 128)
                pltpu.sync_copy(meta_hbm.at[pl.ds(st, 128)],
                                metav.at[pl.ds(0, 128)])

                def ib(j, _):
                    m = metav[pl.ds(j * 16, 16)]
                    srcv = jnp.bitwise_and(m, 0x1FFFF)
                    idxv[pl.ds(j * 16, 16)] = jnp.minimum(srcv, NS - 1)
                    return 0
                lax.fori_loop(0, 8, ib, 0, unroll=True)
                pltpu.async_copy(nm2_hbm.at[idxv], rowsv, sem).wait()
                rem = cnt - kc * 128

                def edge(i, _):
                    m = metav[pl.ds(i, 16)][0]
                    dl = lax.shift_right_logical(m, 18)
                    t = jnp.bitwise_and(lax.shift_right_logical(m, 17), 1)
                    dlv = jnp.where(i < rem, jnp.minimum(dl, TRASH), TRASH)
                    base = dlv * 64
                    for c in range(4):
                        v = rowsv[i, pl.ds(t * 64 + c * 16, 16)]
                        a = accv[pl.ds(base + c * 16, 16)]
                        accv[pl.ds(base + c * 16, 16)] = jnp.maximum(a, v)
                    return 0
                lax.fori_loop(0, 128, edge, 0, unroll=4)
                return 0
            lax.fori_loop(0, nch, chunk, 0)
            off = pl.multiple_of(b * (RB * 64), 128)
            pltpu.sync_copy(accv.at[pl.ds(0, RB * 64)],
                            out_hbm.at[pl.ds(off, RB * 64)])
        return 0
    lax.fori_loop(0, 7, bucket, 0)


def kernel(x, edge_index, edge_attr, edge_type, Wn, bn, We, be, Wr1, Wer1,
           Wroot1, b1, Wr2, Wroot2, b2, Wc, bc):
    N = x.shape[0]
    E = edge_index.shape[1]
    f32 = jnp.float32

    src = edge_index[0]
    dst = edge_index[1]
    dst3 = dst.reshape(NGRID, 1, EB)
    src3 = src.reshape(NGRID, 1, EB)
    typ3 = edge_type.reshape(NGRID, 1, EB)

    hist3, rank3 = pl.pallas_call(
        _hr_body,
        grid=(NGRID,),
        in_specs=[pl.BlockSpec((1, 1, EB), lambda i: (i, 0, 0))],
        out_specs=[pl.BlockSpec((1, 25, NBP), lambda i: (i, 0, 0)),
                   pl.BlockSpec((1, 1, EB), lambda i: (i, 0, 0))],
        out_shape=[jax.ShapeDtypeStruct((NGRID, 25, NBP), f32),
                   jax.ShapeDtypeStruct((NGRID, 1, EB), f32)],
    )(dst3)

    hist = jnp.pad(hist3.reshape(3125, NBP), ((0, 75), (0, 0)))
    co, totI = pl.pallas_call(
        _scan_body,
        grid=(25,),
        in_specs=[pl.BlockSpec((128, NBP), lambda g: (g, 0))],
        out_specs=[pl.BlockSpec((128, NBP), lambda g: (g, 0)),
                   pl.BlockSpec((1, NBP), lambda g: (0, 0))],
        out_shape=[jax.ShapeDtypeStruct((3200, NBP), f32),
                   jax.ShapeDtypeStruct((1, NBP), f32)],
        scratch_shapes=[pltpu.VMEM((8, NBP), f32)],
    )(hist)
    startsI, cntsI = pl.pallas_call(
        _base_body,
        in_specs=[pl.BlockSpec((1, NBP), lambda: (0, 0))],
        out_specs=[pl.BlockSpec((1, 256), lambda: (0, 0)),
                   pl.BlockSpec((1, 256), lambda: (0, 0))],
        out_shape=[jax.ShapeDtypeStruct((1, 256), jnp.int32),
                   jax.ShapeDtypeStruct((1, 256), jnp.int32)],
    )(totI)
    co3 = co[:3125].reshape(NGRID, 25, NBP)

    pos3, meta3 = pl.pallas_call(
        _pos_body,
        grid=(NGRID,),
        in_specs=[pl.BlockSpec((1, 1, EB), lambda i: (i, 0, 0)),
                  pl.BlockSpec((1, 1, EB), lambda i: (i, 0, 0)),
                  pl.BlockSpec((1, 1, EB), lambda i: (i, 0, 0)),
                  pl.BlockSpec((1, 1, EB), lambda i: (i, 0, 0)),
                  pl.BlockSpec((1, 25, NBP), lambda i: (i, 0, 0)),
                  pl.BlockSpec((1, 256), lambda i: (0, 0))],
        out_specs=[pl.BlockSpec((1, 1, EB), lambda i: (i, 0, 0)),
                   pl.BlockSpec((1, 1, EB), lambda i: (i, 0, 0))],
        out_shape=[jax.ShapeDtypeStruct((NGRID, 1, EB), jnp.int32),
                   jax.ShapeDtypeStruct((NGRID, 1, EB), jnp.int32)],
    )(dst3, src3, typ3, rank3, co3, startsI)

    npad = MP2 - E
    pos_pad = jnp.concatenate(
        [pos3.reshape(E), MPOS0 + jnp.arange(npad, dtype=jnp.int32)])
    meta_pad = jnp.concatenate(
        [meta3.reshape(E),
         jnp.full((npad,), TRASH << 18, jnp.int32)])
    a0_pad = jnp.concatenate([edge_attr[:, 0], jnp.zeros((npad,), f32)])
    a1_pad = jnp.concatenate([edge_attr[:, 1], jnp.zeros((npad,), f32)])

    xp = jnp.pad(x, ((0, NS - N), (0, 8 - x.shape[1])))
    Wnp = jnp.pad(Wn, ((0, 8 - Wn.shape[0]), (0, 0)))
    n, nmp = pl.pallas_call(
        _tc1_body,
        grid=(98,),
        in_specs=[pl.BlockSpec((1024, 8), lambda i: (i, 0)),
                  pl.BlockSpec((8, 64), lambda i: (0, 0)),
                  pl.BlockSpec((1, 64), lambda i: (0, 0)),
                  pl.BlockSpec((2, 64, 64), lambda i: (0, 0, 0))],
        out_specs=[pl.BlockSpec((1024, 64), lambda i: (i, 0)),
                   pl.BlockSpec((1024, 128), lambda i: (i, 0))],
        out_shape=[jax.ShapeDtypeStruct((NS, 64), f32),
                   jax.ShapeDtypeStruct((NS, 128), f32)],
    )(xp, Wnp, bn.reshape(1, 64), Wr1)

    metaP, a0P, a1P = pl.kernel(
        _scatter_body,
        out_type=[jax.ShapeDtypeStruct((MP,), jnp.int32),
                  jax.ShapeDtypeStruct((MP,), f32),
                  jax.ShapeDtypeStruct((MP,), f32)],
        mesh=_mesh(),
        scratch_types=[pltpu.VMEM((16, 128), jnp.int32),
                       pltpu.VMEM((16, 128), jnp.int32),
                       pltpu.VMEM((16, 128), f32),
                       pltpu.VMEM((16, 128), f32),
                       pltpu.SemaphoreType.DMA,
                       pltpu.SemaphoreType.DMA,
                       pltpu.SemaphoreType.DMA],
    )(pos_pad.reshape(12800, 128), meta_pad.reshape(12800, 128),
      a0_pad.reshape(12800, 128), a1_pad.reshape(12800, 128))

    wpack = jnp.concatenate(
        [We[0], We[1], be,
         jnp.zeros((32,), f32).at[0].set(1.0)]).reshape(128)
    stpack = jnp.concatenate(
        [startsI.reshape(256, 1), cntsI.reshape(256, 1)], axis=1).reshape(512)
    acc1 = pl.kernel(
        _conv1_body,
        out_type=jax.ShapeDtypeStruct((NS * ACC1W,), f32),
        mesh=_mesh(),
        scratch_types=[pltpu.VMEM(((RB + 1) * ACC1W,), f32),
                       pltpu.VMEM((144,), jnp.int32),
                       pltpu.VMEM((144,), f32),
                       pltpu.VMEM((144,), f32),
                       pltpu.VMEM((128,), jnp.int32),
                       pltpu.VMEM((128, 128), f32),
                       pltpu.VMEM((128,), f32),
                       pltpu.VMEM((528,), jnp.int32),
                       pltpu.SemaphoreType.DMA],
    )(metaP, a0P, a1P, nmp, wpack, stpack,
      jnp.zeros(((RB + 1) * ACC1W,), f32))

    h, nm2p = pl.pallas_call(
        _tc2_body_real,
        grid=(98,),
        in_specs=[pl.BlockSpec((1024, ACC1W), lambda i: (i, 0)),
                  pl.BlockSpec((1024, 64), lambda i: (i, 0)),
                  pl.BlockSpec((2, 32, 64), lambda i: (0, 0, 0)),
                  pl.BlockSpec((64, 64), lambda i: (0, 0)),
                  pl.BlockSpec((1, 64), lambda i: (0, 0)),
                  pl.BlockSpec((2, 64, 64), lambda i: (0, 0, 0))],
        out_specs=[pl.BlockSpec((1024, 64), lambda i: (i, 0)),
                   pl.BlockSpec((1024, 128), lambda i: (i, 0))],
        out_shape=[jax.ShapeDtypeStruct((NS, 64), f32),
                   jax.ShapeDtypeStruct((NS, 128), f32)],
    )(acc1.reshape(NS, ACC1W), n, Wer1, Wroot1, b1.reshape(1, 64), Wr2)

    agg2f = pl.kernel(
        _conv2_body,
        out_type=jax.ShapeDtypeStruct((NS * 64,), f32),
        mesh=_mesh(),
        scratch_types=[pltpu.VMEM(((RB + 1) * 64,), f32),
                       pltpu.VMEM((144,), jnp.int32),
                       pltpu.VMEM((128,), jnp.int32),
                       pltpu.VMEM((128, 128), f32),
                       pltpu.VMEM((528,), jnp.int32),
                       pltpu.SemaphoreType.DMA],
    )(metaP, nm2p, stpack, jnp.full(((RB + 1) * 64,), NEG, f32))

    Wc128 = jnp.pad(Wc, ((0, 0), (0, 127)))
    bc128 = jnp.pad(bc.reshape(1, 1), ((0, 0), (0, 127)))
    out = pl.pallas_call(
        _tc3_body,
        grid=(98,),
        in_specs=[pl.BlockSpec((1024, 64), lambda i: (i, 0)),
                  pl.BlockSpec((1024, 64), lambda i: (i, 0)),
                  pl.BlockSpec((64, 64), lambda i: (0, 0)),
                  pl.BlockSpec((1, 64), lambda i: (0, 0)),
                  pl.BlockSpec((64, 128), lambda i: (0, 0)),
                  pl.BlockSpec((1, 128), lambda i: (0, 0))],
        out_specs=pl.BlockSpec((1024, 128), lambda i: (i, 0)),
        out_shape=jax.ShapeDtypeStruct((NS, 128), f32),
    )(agg2f.reshape(NS, 64), h, Wroot2, b2.reshape(1, 64), Wc128, bc128)
    return out[:N, :1]


def _tc2_body_real(acc_ref, n_ref, wer1_ref, wroot1_ref, b1_ref, wr2_ref,
                   h_ref, nm2_ref):
    acc = acc_ref[...]
    nm_s = acc[:, 0:64]
    e0 = acc[:, 64:96]
    c0 = acc[:, 96:97]
    e1 = acc[:, 112:144]
    c1 = acc[:, 144:145]
    deg = jnp.maximum(c0 + c1, 1.0)
    agg = (nm_s
           + jnp.dot(e0, wer1_ref[0], preferred_element_type=jnp.float32, precision=lax.Precision.HIGHEST)
           + jnp.dot(e1, wer1_ref[1], preferred_element_type=jnp.float32, precision=lax.Precision.HIGHEST)
           ) / deg
    hv = jax.nn.relu(
        agg + jnp.dot(n_ref[...], wroot1_ref[...],
                      preferred_element_type=jnp.float32,
                      precision=lax.Precision.HIGHEST) + b1_ref[...])
    h_ref[...] = hv
    nm2_ref[...] = jnp.concatenate(
        [jnp.dot(hv, wr2_ref[0], preferred_element_type=jnp.float32, precision=lax.Precision.HIGHEST),
         jnp.dot(hv, wr2_ref[1], preferred_element_type=jnp.float32, precision=lax.Precision.HIGHEST)], axis=1)


# speedup vs baseline: 4.3555x; 1.0842x over previous
import functools

import jax
import jax.numpy as jnp
from jax import lax
from jax.experimental import pallas as pl
from jax.experimental.pallas import tpu as pltpu
from jax.experimental.pallas import tpu_sc as plsc

NS = 100352
RB = 512
NBUCK = 196
NBP = 224
CH = 512
EB = 12800
NGRID = 125
TRASH = 512
MP = 1663488
MPOS0 = 1625088
MP2 = 1638400
ACC1W = 160
NEG = -3.0e38


def _hr_body(dst_ref, hist_ref, rank_ref):
    d = dst_ref[0, 0, :]
    b = lax.shift_right_logical(d, 9).reshape(25, CH)
    iota_b = lax.broadcasted_iota(jnp.int32, (1, NBP), 1)
    lt = lax.broadcasted_iota(jnp.int32, (CH, CH), 0) > lax.broadcasted_iota(
        jnp.int32, (CH, CH), 1)
    L = lt.astype(jnp.bfloat16)
    ranks = []
    hists = []
    for c in range(25):
        M = (b[c][:, None] == iota_b).astype(jnp.float32)
        hists.append(jnp.sum(M, axis=0))
        C = lax.dot_general(L, M.astype(jnp.bfloat16), (((1,), (0,)), ((), ())),
                            preferred_element_type=jnp.float32)
        ranks.append(jnp.sum(M * C, axis=1))
    hist_ref[0] = jnp.stack(hists)
    rank_ref[0, 0] = jnp.concatenate(ranks)


def _scan_body(hist_ref, co_ref, tot_ref, carry_ref):
    g = pl.program_id(0)

    @pl.when(g == 0)
    def _():
        carry_ref[...] = jnp.zeros((8, NBP), jnp.float32)

    h = hist_ref[...]
    lt128 = (lax.broadcasted_iota(jnp.int32, (128, 128), 0)
             > lax.broadcasted_iota(jnp.int32, (128, 128), 1))
    L128 = lt128.astype(jnp.bfloat16)
    within = lax.dot_general(L128, h.astype(jnp.bfloat16),
                             (((1,), (0,)), ((), ())),
                             preferred_element_type=jnp.float32)
    carry = carry_ref[...]
    co_ref[...] = within + carry[0:1, :]
    s = jnp.sum(h, axis=0, keepdims=True)
    carry2 = carry + jnp.broadcast_to(s, (8, NBP))
    carry_ref[...] = carry2
    tot_ref[...] = carry2[0:1, :]


def _base_body(tot_ref, starts_ref, cnts_ref):
    tot = tot_ref[0, :]
    sz = jnp.ceil(tot * (1.0 / 128.0)) * 128.0
    ltB = (lax.broadcasted_iota(jnp.int32, (NBP, NBP), 0)
           > lax.broadcasted_iota(jnp.int32, (NBP, NBP), 1)).astype(jnp.float32)
    base = jnp.sum(ltB * sz[None, :], axis=1)
    sp = jnp.concatenate([base, jnp.full((32,), float(MPOS0), jnp.float32)])
    starts_ref[...] = sp.astype(jnp.int32).reshape(1, 256)
    cnts_ref[...] = jnp.concatenate(
        [tot, jnp.zeros((32,), jnp.float32)]).astype(jnp.int32).reshape(1, 256)


def _pos_body(dst_ref, src_ref, typ_ref, rank_ref, co_ref, starts_ref,
              pos_ref, meta_ref):
    d = dst_ref[0, 0, :]
    b = lax.shift_right_logical(d, 9).reshape(25, CH)
    dl = jnp.bitwise_and(d, RB - 1)
    iota_b = lax.broadcasted_iota(jnp.int32, (1, NBP), 1)
    basef = starts_ref[0, 0:NBP].astype(jnp.float32)
    offs = []
    for c in range(25):
        M = (b[c][:, None] == iota_b).astype(jnp.float32)
        offs.append(jnp.sum(M * (co_ref[0, c] + basef)[None, :], axis=1))
    pos = jnp.concatenate(offs) + rank_ref[0, 0]
    pos_ref[0, 0] = pos.astype(jnp.int32)
    meta_ref[0, 0] = (src_ref[0, 0, :]
                      + lax.shift_left(typ_ref[0, 0, :], 17)
                      + lax.shift_left(dl, 18))


def _tc1_body(x_ref, wn_ref, bn_ref, wr1_ref, n_ref, nm_ref):
    nv = jax.nn.relu(
        jnp.dot(x_ref[...], wn_ref[...], preferred_element_type=jnp.float32, precision=lax.Precision.HIGHEST)
        + bn_ref[...])
    n_ref[...] = nv
    nm_ref[...] = jnp.concatenate(
        [jnp.dot(nv, wr1_ref[0], preferred_element_type=jnp.float32, precision=lax.Precision.HIGHEST),
         jnp.dot(nv, wr1_ref[1], preferred_element_type=jnp.float32, precision=lax.Precision.HIGHEST)], axis=1)


def _tc3_body(agg2_ref, h_ref, wroot2_ref, b2_ref, wc_ref, bc_ref, o_ref):
    a2 = agg2_ref[...]
    a2 = jnp.where(a2 > NEG, a2, 0.0)
    h2 = jax.nn.relu(
        a2 + jnp.dot(h_ref[...], wroot2_ref[...],
                     preferred_element_type=jnp.float32,
                     precision=lax.Precision.HIGHEST) + b2_ref[...])
    o_ref[...] = jnp.tanh(
        jnp.dot(h2, wc_ref[...], preferred_element_type=jnp.float32,
                precision=lax.Precision.HIGHEST)
        + bc_ref[...]) * 5.0


def _mesh():
    return plsc.VectorSubcoreMesh(core_axis_name="c", subcore_axis_name="s")


def _wid():
    return lax.axis_index("s") * 2 + lax.axis_index("c")


def _scatter_body(pos_hbm, meta_hbm, a0_hbm, a1_hbm,
                  metaP_hbm, a0P_hbm, a1P_hbm,
                  posv, mv, a0v, a1v, sem, sem2, sem3):
    wid = _wid()
    base = wid * 25 * 16
    def chunk(k, _):
        row = base + k * 16
        pltpu.sync_copy(pos_hbm.at[pl.ds(row, 16)], posv)
        pltpu.sync_copy(meta_hbm.at[pl.ds(row, 16)], mv)
        pltpu.sync_copy(a0_hbm.at[pl.ds(row, 16)], a0v)
        pltpu.sync_copy(a1_hbm.at[pl.ds(row, 16)], a1v)
        cps = []
        for j in range(16):
            cps.append(pltpu.async_copy(
                mv.at[j], metaP_hbm.at[posv.at[j]], sem))
            cps.append(pltpu.async_copy(
                a0v.at[j], a0P_hbm.at[posv.at[j]], sem2))
            cps.append(pltpu.async_copy(
                a1v.at[j], a1P_hbm.at[posv.at[j]], sem3))
        for cp in cps:
            cp.wait()
        return 0
    lax.fori_loop(0, 25, chunk, 0)


def _conv1_body(meta_hbm, a0_hbm, a1_hbm, nmp_hbm, wpack_hbm, st_hbm,
                zero_hbm,
                out_hbm,
                accv, metav, a0v, a1v, idxv, rowsv, wv, stv, sem):
    wid = _wid()
    pltpu.sync_copy(wpack_hbm, wv)
    pltpu.sync_copy(st_hbm, stv.at[pl.ds(0, 512)])
    w00 = wv[pl.ds(0, 16)]
    w01 = wv[pl.ds(16, 16)]
    w10 = wv[pl.ds(32, 16)]
    w11 = wv[pl.ds(48, 16)]
    wb0 = wv[pl.ds(64, 16)]
    wb1 = wv[pl.ds(80, 16)]
    cunit = wv[pl.ds(96, 16)]

    def bucket(kb, _):
        b = wid + kb * 32

        @pl.when(b < NBUCK)
        def _():
            sv = stv[pl.ds(b * 2, 16)]
            start = pl.multiple_of(sv[0], 128)
            cnt = sv[1]
            pltpu.sync_copy(zero_hbm, accv)
            nch = lax.div(cnt + 127, 128)

            def loadfire(kc, p):
                st = pl.multiple_of(start + kc * 128, 128)
                pltpu.sync_copy(meta_hbm.at[pl.ds(st, 128)],
                                metav.at[pl.ds(p * 144, 128)])
                pltpu.sync_copy(a0_hbm.at[pl.ds(st, 128)],
                                a0v.at[pl.ds(p * 144, 128)])
                pltpu.sync_copy(a1_hbm.at[pl.ds(st, 128)],
                                a1v.at[pl.ds(p * 144, 128)])

                def ib(j, _):
                    m = metav[pl.ds(p * 144 + j * 16, 16)]
                    srcv = jnp.bitwise_and(m, 0x1FFFF)
                    idxv[p, pl.ds(j * 16, 16)] = jnp.minimum(srcv, NS - 1)
                    return 0
                lax.fori_loop(0, 8, ib, 0, unroll=True)
                pltpu.async_copy(nmp_hbm.at[idxv.at[p]], rowsv.at[p], sem)

            def process(kc, p):
                rem = cnt - kc * 128

                def edge(i, _):
                    m = metav[pl.ds(p * 144 + i, 16)][0]
                    dl = lax.shift_right_logical(m, 18)
                    t = jnp.bitwise_and(lax.shift_right_logical(m, 17), 1)
                    dlv = jnp.where(i < rem, jnp.minimum(dl, TRASH), TRASH)
                    base = dlv * ACC1W
                    for c in range(4):
                        v = rowsv[p, i, pl.ds(t * 64 + c * 16, 16)]
                        plsc.addupdate(accv.at[pl.ds(base + c * 16, 16)], v)
                    a0 = a0v[pl.ds(p * 144 + i, 16)][0]
                    a1 = a1v[pl.ds(p * 144 + i, 16)][0]
                    eb = base + 64 + t * 48
                    ef0 = jnp.maximum(a0 * w00 + a1 * w10 + wb0, 0.0)
                    plsc.addupdate(accv.at[pl.ds(eb, 16)], ef0)
                    ef1 = jnp.maximum(a0 * w01 + a1 * w11 + wb1, 0.0)
                    plsc.addupdate(accv.at[pl.ds(eb + 16, 16)], ef1)
                    plsc.addupdate(accv.at[pl.ds(eb + 32, 16)], cunit)
                    return 0
                lax.fori_loop(0, 128, edge, 0, unroll=4)

            def chunk(kc, _):
                p = jnp.bitwise_and(kc, 1)
                loadfire(kc, p)

                @pl.when(kc > 0)
                def _():
                    pm = 1 - p
                    pltpu.make_async_copy(
                        nmp_hbm.at[idxv.at[pm]], rowsv.at[pm], sem).wait()
                    process(kc - 1, pm)
                return 0
            lax.fori_loop(0, nch, chunk, 0)

            @pl.when(nch > 0)
            def _():
                lastp = jnp.bitwise_and(nch - 1, 1)
                pltpu.make_async_copy(
                    nmp_hbm.at[idxv.at[lastp]], rowsv.at[lastp], sem).wait()
                process(nch - 1, lastp)
            off = pl.multiple_of(b * (RB * ACC1W), 128)
            pltpu.sync_copy(accv.at[pl.ds(0, RB * ACC1W)],
                            out_hbm.at[pl.ds(off, RB * ACC1W)])
        return 0
    lax.fori_loop(0, 7, bucket, 0)


def _conv2_body(meta_hbm, nm2_hbm, st_hbm, neg_hbm,
                out_hbm,
                accv, metav, idxv, rowsv, stv, sem):
    wid = _wid()
    pltpu.sync_copy(st_hbm, stv.at[pl.ds(0, 512)])

    def bucket(kb, _):
        b = wid + kb * 32

        @pl.when(b < NBUCK)
        def _():
            sv = stv[pl.ds(b * 2, 16)]
            start = pl.multiple_of(sv[0], 128)
            cnt = sv[1]
            pltpu.sync_copy(neg_hbm, accv)
            nch = lax.div(cnt + 127, 128)

            def loadfire(kc, p):
                st = pl.multiple_of(start + kc * 128, 128)
                pltpu.sync_copy(meta_hbm.at[pl.ds(st, 128)],
                                metav.at[pl.ds(p * 144, 128)])

                def ib(j, _):
                    m = metav[pl.ds(p * 144 + j * 16, 16)]
                    srcv = jnp.bitwise_and(m, 0x1FFFF)
                    idxv[p, pl.ds(j * 16, 16)] = jnp.minimum(srcv, NS - 1)
                    return 0
                lax.fori_loop(0, 8, ib, 0, unroll=True)
                pltpu.async_copy(nm2_hbm.at[idxv.at[p]], rowsv.at[p], sem)

            def process(kc, p):
                rem = cnt - kc * 128

                def edge(i, _):
                    m = metav[pl.ds(p * 144 + i, 16)][0]
                    dl = lax.shift_right_logical(m, 18)
                    t = jnp.bitwise_and(lax.shift_right_logical(m, 17), 1)
                    dlv = jnp.where(i < rem, jnp.minimum(dl, TRASH), TRASH)
                    base = dlv * 64
                    for c in range(4):
                        v = rowsv[p, i, pl.ds(t * 64 + c * 16, 16)]
                        a = accv[pl.ds(base + c * 16, 16)]
                        accv[pl.ds(base + c * 16, 16)] = jnp.maximum(a, v)
                    return 0
                lax.fori_loop(0, 128, edge, 0, unroll=4)

            def chunk(kc, _):
                p = jnp.bitwise_and(kc, 1)
                loadfire(kc, p)

                @pl.when(kc > 0)
                def _():
                    pm = 1 - p
                    pltpu.make_async_copy(
                        nm2_hbm.at[idxv.at[pm]], rowsv.at[pm], sem).wait()
                    process(kc - 1, pm)
                return 0
            lax.fori_loop(0, nch, chunk, 0)

            @pl.when(nch > 0)
            def _():
                lastp = jnp.bitwise_and(nch - 1, 1)
                pltpu.make_async_copy(
                    nm2_hbm.at[idxv.at[lastp]], rowsv.at[lastp], sem).wait()
                process(nch - 1, lastp)
            off = pl.multiple_of(b * (RB * 64), 128)
            pltpu.sync_copy(accv.at[pl.ds(0, RB * 64)],
                            out_hbm.at[pl.ds(off, RB * 64)])
        return 0
    lax.fori_loop(0, 7, bucket, 0)


def kernel(x, edge_index, edge_attr, edge_type, Wn, bn, We, be, Wr1, Wer1,
           Wroot1, b1, Wr2, Wroot2, b2, Wc, bc):
    N = x.shape[0]
    E = edge_index.shape[1]
    f32 = jnp.float32

    src = edge_index[0]
    dst = edge_index[1]
    dst3 = dst.reshape(NGRID, 1, EB)
    src3 = src.reshape(NGRID, 1, EB)
    typ3 = edge_type.reshape(NGRID, 1, EB)

    hist3, rank3 = pl.pallas_call(
        _hr_body,
        grid=(NGRID,),
        in_specs=[pl.BlockSpec((1, 1, EB), lambda i: (i, 0, 0))],
        out_specs=[pl.BlockSpec((1, 25, NBP), lambda i: (i, 0, 0)),
                   pl.BlockSpec((1, 1, EB), lambda i: (i, 0, 0))],
        out_shape=[jax.ShapeDtypeStruct((NGRID, 25, NBP), f32),
                   jax.ShapeDtypeStruct((NGRID, 1, EB), f32)],
    )(dst3)

    hist = jnp.pad(hist3.reshape(3125, NBP), ((0, 75), (0, 0)))
    co, totI = pl.pallas_call(
        _scan_body,
        grid=(25,),
        in_specs=[pl.BlockSpec((128, NBP), lambda g: (g, 0))],
        out_specs=[pl.BlockSpec((128, NBP), lambda g: (g, 0)),
                   pl.BlockSpec((1, NBP), lambda g: (0, 0))],
        out_shape=[jax.ShapeDtypeStruct((3200, NBP), f32),
                   jax.ShapeDtypeStruct((1, NBP), f32)],
        scratch_shapes=[pltpu.VMEM((8, NBP), f32)],
    )(hist)
    startsI, cntsI = pl.pallas_call(
        _base_body,
        in_specs=[pl.BlockSpec((1, NBP), lambda: (0, 0))],
        out_specs=[pl.BlockSpec((1, 256), lambda: (0, 0)),
                   pl.BlockSpec((1, 256), lambda: (0, 0))],
        out_shape=[jax.ShapeDtypeStruct((1, 256), jnp.int32),
                   jax.ShapeDtypeStruct((1, 256), jnp.int32)],
    )(totI)
    co3 = co[:3125].reshape(NGRID, 25, NBP)

    pos3, meta3 = pl.pallas_call(
        _pos_body,
        grid=(NGRID,),
        in_specs=[pl.BlockSpec((1, 1, EB), lambda i: (i, 0, 0)),
                  pl.BlockSpec((1, 1, EB), lambda i: (i, 0, 0)),
                  pl.BlockSpec((1, 1, EB), lambda i: (i, 0, 0)),
                  pl.BlockSpec((1, 1, EB), lambda i: (i, 0, 0)),
                  pl.BlockSpec((1, 25, NBP), lambda i: (i, 0, 0)),
                  pl.BlockSpec((1, 256), lambda i: (0, 0))],
        out_specs=[pl.BlockSpec((1, 1, EB), lambda i: (i, 0, 0)),
                   pl.BlockSpec((1, 1, EB), lambda i: (i, 0, 0))],
        out_shape=[jax.ShapeDtypeStruct((NGRID, 1, EB), jnp.int32),
                   jax.ShapeDtypeStruct((NGRID, 1, EB), jnp.int32)],
    )(dst3, src3, typ3, rank3, co3, startsI)

    npad = MP2 - E
    pos_pad = jnp.concatenate(
        [pos3.reshape(E), MPOS0 + jnp.arange(npad, dtype=jnp.int32)])
    meta_pad = jnp.concatenate(
        [meta3.reshape(E),
         jnp.full((npad,), TRASH << 18, jnp.int32)])
    a0_pad = jnp.concatenate([edge_attr[:, 0], jnp.zeros((npad,), f32)])
    a1_pad = jnp.concatenate([edge_attr[:, 1], jnp.zeros((npad,), f32)])

    xp = jnp.pad(x, ((0, NS - N), (0, 8 - x.shape[1])))
    Wnp = jnp.pad(Wn, ((0, 8 - Wn.shape[0]), (0, 0)))
    n, nmp = pl.pallas_call(
        _tc1_body,
        grid=(98,),
        in_specs=[pl.BlockSpec((1024, 8), lambda i: (i, 0)),
                  pl.BlockSpec((8, 64), lambda i: (0, 0)),
                  pl.BlockSpec((1, 64), lambda i: (0, 0)),
                  pl.BlockSpec((2, 64, 64), lambda i: (0, 0, 0))],
        out_specs=[pl.BlockSpec((1024, 64), lambda i: (i, 0)),
                   pl.BlockSpec((1024, 128), lambda i: (i, 0))],
        out_shape=[jax.ShapeDtypeStruct((NS, 64), f32),
                   jax.ShapeDtypeStruct((NS, 128), f32)],
    )(xp, Wnp, bn.reshape(1, 64), Wr1)

    metaP, a0P, a1P = pl.kernel(
        _scatter_body,
        out_type=[jax.ShapeDtypeStruct((MP,), jnp.int32),
                  jax.ShapeDtypeStruct((MP,), f32),
                  jax.ShapeDtypeStruct((MP,), f32)],
        mesh=_mesh(),
        scratch_types=[pltpu.VMEM((16, 128), jnp.int32),
                       pltpu.VMEM((16, 128), jnp.int32),
                       pltpu.VMEM((16, 128), f32),
                       pltpu.VMEM((16, 128), f32),
                       pltpu.SemaphoreType.DMA,
                       pltpu.SemaphoreType.DMA,
                       pltpu.SemaphoreType.DMA],
    )(pos_pad.reshape(12800, 128), meta_pad.reshape(12800, 128),
      a0_pad.reshape(12800, 128), a1_pad.reshape(12800, 128))

    wpack = jnp.concatenate(
        [We[0], We[1], be,
         jnp.zeros((32,), f32).at[0].set(1.0)]).reshape(128)
    stpack = jnp.concatenate(
        [startsI.reshape(256, 1), cntsI.reshape(256, 1)], axis=1).reshape(512)
    acc1 = pl.kernel(
        _conv1_body,
        out_type=jax.ShapeDtypeStruct((NS * ACC1W,), f32),
        mesh=_mesh(),
        scratch_types=[pltpu.VMEM(((RB + 1) * ACC1W,), f32),
                       pltpu.VMEM((304,), jnp.int32),
                       pltpu.VMEM((304,), f32),
                       pltpu.VMEM((304,), f32),
                       pltpu.VMEM((2, 128), jnp.int32),
                       pltpu.VMEM((2, 128, 128), f32),
                       pltpu.VMEM((128,), f32),
                       pltpu.VMEM((528,), jnp.int32),
                       pltpu.SemaphoreType.DMA],
    )(metaP, a0P, a1P, nmp, wpack, stpack,
      jnp.zeros(((RB + 1) * ACC1W,), f32))

    h, nm2p = pl.pallas_call(
        _tc2_body_real,
        grid=(98,),
        in_specs=[pl.BlockSpec((1024, ACC1W), lambda i: (i, 0)),
                  pl.BlockSpec((1024, 64), lambda i: (i, 0)),
                  pl.BlockSpec((2, 32, 64), lambda i: (0, 0, 0)),
                  pl.BlockSpec((64, 64), lambda i: (0, 0)),
                  pl.BlockSpec((1, 64), lambda i: (0, 0)),
                  pl.BlockSpec((2, 64, 64), lambda i: (0, 0, 0))],
        out_specs=[pl.BlockSpec((1024, 64), lambda i: (i, 0)),
                   pl.BlockSpec((1024, 128), lambda i: (i, 0))],
        out_shape=[jax.ShapeDtypeStruct((NS, 64), f32),
                   jax.ShapeDtypeStruct((NS, 128), f32)],
    )(acc1.reshape(NS, ACC1W), n, Wer1, Wroot1, b1.reshape(1, 64), Wr2)

    agg2f = pl.kernel(
        _conv2_body,
        out_type=jax.ShapeDtypeStruct((NS * 64,), f32),
        mesh=_mesh(),
        scratch_types=[pltpu.VMEM(((RB + 1) * 64,), f32),
                       pltpu.VMEM((304,), jnp.int32),
                       pltpu.VMEM((2, 128), jnp.int32),
                       pltpu.VMEM((2, 128, 128), f32),
                       pltpu.VMEM((528,), jnp.int32),
                       pltpu.SemaphoreType.DMA],
    )(metaP, nm2p, stpack, jnp.full(((RB + 1) * 64,), NEG, f32))

    Wc128 = jnp.pad(Wc, ((0, 0), (0, 127)))
    bc128 = jnp.pad(bc.reshape(1, 1), ((0, 0), (0, 127)))
    out = pl.pallas_call(
        _tc3_body,
        grid=(98,),
        in_specs=[pl.BlockSpec((1024, 64), lambda i: (i, 0)),
                  pl.BlockSpec((1024, 64), lambda i: (i, 0)),
                  pl.BlockSpec((64, 64), lambda i: (0, 0)),
                  pl.BlockSpec((1, 64), lambda i: (0, 0)),
                  pl.BlockSpec((64, 128), lambda i: (0, 0)),
                  pl.BlockSpec((1, 128), lambda i: (0, 0))],
        out_specs=pl.BlockSpec((1024, 128), lambda i: (i, 0)),
        out_shape=jax.ShapeDtypeStruct((NS, 128), f32),
    )(agg2f.reshape(NS, 64), h, Wroot2, b2.reshape(1, 64), Wc128, bc128)
    return out[:N, :1]


def _tc2_body_real(acc_ref, n_ref, wer1_ref, wroot1_ref, b1_ref, wr2_ref,
                   h_ref, nm2_ref):
    acc = acc_ref[...]
    nm_s = acc[:, 0:64]
    e0 = acc[:, 64:96]
    c0 = acc[:, 96:97]
    e1 = acc[:, 112:144]
    c1 = acc[:, 144:145]
    deg = jnp.maximum(c0 + c1, 1.0)
    agg = (nm_s
           + jnp.dot(e0, wer1_ref[0], preferred_element_type=jnp.float32, precision=lax.Precision.HIGHEST)
           + jnp.dot(e1, wer1_ref[1], preferred_element_type=jnp.float32, precision=lax.Precision.HIGHEST)
           ) / deg
    hv = jax.nn.relu(
        agg + jnp.dot(n_ref[...], wroot1_ref[...],
                      preferred_element_type=jnp.float32,
                      precision=lax.Precision.HIGHEST) + b1_ref[...])
    h_ref[...] = hv
    nm2_ref[...] = jnp.concatenate(
        [jnp.dot(hv, wr2_ref[0], preferred_element_type=jnp.float32, precision=lax.Precision.HIGHEST),
         jnp.dot(hv, wr2_ref[1], preferred_element_type=jnp.float32, precision=lax.Precision.HIGHEST)], axis=1)


# speedup vs baseline: 4.3558x; 1.0001x over previous
import functools

import jax
import jax.numpy as jnp
from jax import lax
from jax.experimental import pallas as pl
from jax.experimental.pallas import tpu as pltpu
from jax.experimental.pallas import tpu_sc as plsc

NS = 100352
RB = 512
NBUCK = 196
NBP = 224
CH = 512
EB = 12800
NGRID = 125
TRASH = 512
MP = 1663488
MPOS0 = 1625088
MP2 = 1638400
ACC1W = 160
NEG = -3.0e38


def _hr_body(dst_ref, hist_ref, rank_ref):
    d = dst_ref[0, 0, :]
    b = lax.shift_right_logical(d, 9).reshape(25, CH)
    iota_b = lax.broadcasted_iota(jnp.int32, (1, NBP), 1)
    lt = lax.broadcasted_iota(jnp.int32, (CH, CH), 0) > lax.broadcasted_iota(
        jnp.int32, (CH, CH), 1)
    L = lt.astype(jnp.bfloat16)
    ranks = []
    hists = []
    for c in range(25):
        M = (b[c][:, None] == iota_b).astype(jnp.float32)
        hists.append(jnp.sum(M, axis=0))
        C = lax.dot_general(L, M.astype(jnp.bfloat16), (((1,), (0,)), ((), ())),
                            preferred_element_type=jnp.float32)
        ranks.append(jnp.sum(M * C, axis=1))
    hist_ref[0] = jnp.stack(hists)
    rank_ref[0, 0] = jnp.concatenate(ranks)


def _scan_body(hist_ref, co_ref, tot_ref, carry_ref):
    g = pl.program_id(0)

    @pl.when(g == 0)
    def _():
        carry_ref[...] = jnp.zeros((8, NBP), jnp.float32)

    h = hist_ref[...]
    lt128 = (lax.broadcasted_iota(jnp.int32, (128, 128), 0)
             > lax.broadcasted_iota(jnp.int32, (128, 128), 1))
    L128 = lt128.astype(jnp.bfloat16)
    within = lax.dot_general(L128, h.astype(jnp.bfloat16),
                             (((1,), (0,)), ((), ())),
                             preferred_element_type=jnp.float32)
    carry = carry_ref[...]
    co_ref[...] = within + carry[0:1, :]
    s = jnp.sum(h, axis=0, keepdims=True)
    carry2 = carry + jnp.broadcast_to(s, (8, NBP))
    carry_ref[...] = carry2
    tot_ref[...] = carry2[0:1, :]


def _base_body(tot_ref, starts_ref, cnts_ref):
    tot = tot_ref[0, :]
    sz = jnp.ceil(tot * (1.0 / 128.0)) * 128.0
    ltB = (lax.broadcasted_iota(jnp.int32, (NBP, NBP), 0)
           > lax.broadcasted_iota(jnp.int32, (NBP, NBP), 1)).astype(jnp.float32)
    base = jnp.sum(ltB * sz[None, :], axis=1)
    sp = jnp.concatenate([base, jnp.full((32,), float(MPOS0), jnp.float32)])
    starts_ref[...] = sp.astype(jnp.int32).reshape(1, 256)
    cnts_ref[...] = jnp.concatenate(
        [tot, jnp.zeros((32,), jnp.float32)]).astype(jnp.int32).reshape(1, 256)


def _pos_body(dst_ref, src_ref, typ_ref, rank_ref, co_ref, starts_ref,
              pos_ref, meta_ref):
    d = dst_ref[0, 0, :]
    b = lax.shift_right_logical(d, 9).reshape(25, CH)
    dl = jnp.bitwise_and(d, RB - 1)
    iota_b = lax.broadcasted_iota(jnp.int32, (1, NBP), 1)
    basef = starts_ref[0, 0:NBP].astype(jnp.float32)
    offs = []
    for c in range(25):
        M = (b[c][:, None] == iota_b).astype(jnp.float32)
        offs.append(jnp.sum(M * (co_ref[0, c] + basef)[None, :], axis=1))
    pos = jnp.concatenate(offs) + rank_ref[0, 0]
    pos_ref[0, 0] = pos.astype(jnp.int32)
    meta_ref[0, 0] = (src_ref[0, 0, :]
                      + lax.shift_left(typ_ref[0, 0, :], 17)
                      + lax.shift_left(dl, 18))


def _tc1_body(x_ref, wn_ref, bn_ref, wr1_ref, n_ref, nm_ref):
    nv = jax.nn.relu(
        jnp.dot(x_ref[...], wn_ref[...], preferred_element_type=jnp.float32, precision=lax.Precision.HIGHEST)
        + bn_ref[...])
    n_ref[...] = nv
    nm_ref[...] = jnp.concatenate(
        [jnp.dot(nv, wr1_ref[0], preferred_element_type=jnp.float32, precision=lax.Precision.HIGHEST),
         jnp.dot(nv, wr1_ref[1], preferred_element_type=jnp.float32, precision=lax.Precision.HIGHEST)], axis=1)


def _tc3_body(agg2_ref, h_ref, wroot2_ref, b2_ref, wc_ref, bc_ref, o_ref):
    a2 = agg2_ref[...]
    a2 = jnp.where(a2 > NEG, a2, 0.0)
    h2 = jax.nn.relu(
        a2 + jnp.dot(h_ref[...], wroot2_ref[...],
                     preferred_element_type=jnp.float32,
                     precision=lax.Precision.HIGHEST) + b2_ref[...])
    o_ref[...] = jnp.tanh(
        jnp.dot(h2, wc_ref[...], preferred_element_type=jnp.float32,
                precision=lax.Precision.HIGHEST)
        + bc_ref[...]) * 5.0


def _mesh():
    return plsc.VectorSubcoreMesh(core_axis_name="c", subcore_axis_name="s")


def _wid():
    return lax.axis_index("s") * 2 + lax.axis_index("c")


def _scatter_body(pos_hbm, meta_hbm, a0_hbm, a1_hbm,
                  metaP_hbm, a0P_hbm, a1P_hbm,
                  posv, mv, a0v, a1v, semm0, semm1, sema0, sema1, semb0, semb1):
    wid = _wid()
    base = wid * 25 * 16
    semm = (semm0, semm1)
    sema = (sema0, sema1)
    semb = (semb0, semb1)

    def fire(p):
        cps = []
        for j in range(16):
            cps.append(pltpu.async_copy(
                mv.at[p, j], metaP_hbm.at[posv.at[p, j]], semm[p]))
            cps.append(pltpu.async_copy(
                a0v.at[p, j], a0P_hbm.at[posv.at[p, j]], sema[p]))
            cps.append(pltpu.async_copy(
                a1v.at[p, j], a1P_hbm.at[posv.at[p, j]], semb[p]))
        return cps

    def drain(p):
        for j in range(16):
            pltpu.make_async_copy(
                mv.at[p, j], metaP_hbm.at[posv.at[p, j]], semm[p]).wait()
            pltpu.make_async_copy(
                a0v.at[p, j], a0P_hbm.at[posv.at[p, j]], sema[p]).wait()
            pltpu.make_async_copy(
                a1v.at[p, j], a1P_hbm.at[posv.at[p, j]], semb[p]).wait()

    def chunk(k, _):
        row = base + k * 16
        p = k % 2
        if k >= 2:
            drain(p)
        pltpu.sync_copy(pos_hbm.at[pl.ds(row, 16)], posv.at[p])
        pltpu.sync_copy(meta_hbm.at[pl.ds(row, 16)], mv.at[p])
        pltpu.sync_copy(a0_hbm.at[pl.ds(row, 16)], a0v.at[p])
        pltpu.sync_copy(a1_hbm.at[pl.ds(row, 16)], a1v.at[p])
        fire(p)
        return 0

    for k in range(25):
        chunk(k, 0)
    drain(1)
    drain(0)


def _conv1_body(meta_hbm, a0_hbm, a1_hbm, nmp_hbm, wpack_hbm, st_hbm,
                zero_hbm,
                out_hbm,
                accv, metav, a0v, a1v, idxv, rowsv, wv, stv, sem):
    wid = _wid()
    pltpu.sync_copy(wpack_hbm, wv)
    pltpu.sync_copy(st_hbm, stv.at[pl.ds(0, 512)])
    w00 = wv[pl.ds(0, 16)]
    w01 = wv[pl.ds(16, 16)]
    w10 = wv[pl.ds(32, 16)]
    w11 = wv[pl.ds(48, 16)]
    wb0 = wv[pl.ds(64, 16)]
    wb1 = wv[pl.ds(80, 16)]
    cunit = wv[pl.ds(96, 16)]

    def bucket(kb, _):
        b = wid + kb * 32

        @pl.when(b < NBUCK)
        def _():
            sv = stv[pl.ds(b * 2, 16)]
            start = pl.multiple_of(sv[0], 128)
            cnt = sv[1]
            pltpu.sync_copy(zero_hbm, accv)
            nch = lax.div(cnt + 127, 128)

            def loadfire(kc, p):
                st = pl.multiple_of(start + kc * 128, 128)
                pltpu.sync_copy(meta_hbm.at[pl.ds(st, 128)],
                                metav.at[pl.ds(p * 144, 128)])
                pltpu.sync_copy(a0_hbm.at[pl.ds(st, 128)],
                                a0v.at[pl.ds(p * 144, 128)])
                pltpu.sync_copy(a1_hbm.at[pl.ds(st, 128)],
                                a1v.at[pl.ds(p * 144, 128)])

                def ib(j, _):
                    m = metav[pl.ds(p * 144 + j * 16, 16)]
                    srcv = jnp.bitwise_and(m, 0x1FFFF)
                    idxv[p, pl.ds(j * 16, 16)] = jnp.minimum(srcv, NS - 1)
                    return 0
                lax.fori_loop(0, 8, ib, 0, unroll=True)
                pltpu.async_copy(nmp_hbm.at[idxv.at[p]], rowsv.at[p], sem)

            def process(kc, p):
                rem = cnt - kc * 128

                def edge(i, _):
                    m = metav[pl.ds(p * 144 + i, 16)][0]
                    dl = lax.shift_right_logical(m, 18)
                    t = jnp.bitwise_and(lax.shift_right_logical(m, 17), 1)
                    dlv = jnp.where(i < rem, jnp.minimum(dl, TRASH), TRASH)
                    base = dlv * ACC1W
                    for c in range(4):
                        v = rowsv[p, i, pl.ds(t * 64 + c * 16, 16)]
                        plsc.addupdate(accv.at[pl.ds(base + c * 16, 16)], v)
                    a0 = a0v[pl.ds(p * 144 + i, 16)][0]
                    a1 = a1v[pl.ds(p * 144 + i, 16)][0]
                    eb = base + 64 + t * 48
                    ef0 = jnp.maximum(a0 * w00 + a1 * w10 + wb0, 0.0)
                    plsc.addupdate(accv.at[pl.ds(eb, 16)], ef0)
                    ef1 = jnp.maximum(a0 * w01 + a1 * w11 + wb1, 0.0)
                    plsc.addupdate(accv.at[pl.ds(eb + 16, 16)], ef1)
                    plsc.addupdate(accv.at[pl.ds(eb + 32, 16)], cunit)
                    return 0
                lax.fori_loop(0, 128, edge, 0, unroll=4)

            def chunk(kc, _):
                p = jnp.bitwise_and(kc, 1)
                loadfire(kc, p)

                @pl.when(kc > 0)
                def _():
                    pm = 1 - p
                    pltpu.make_async_copy(
                        nmp_hbm.at[idxv.at[pm]], rowsv.at[pm], sem).wait()
                    process(kc - 1, pm)
                return 0
            lax.fori_loop(0, nch, chunk, 0)

            @pl.when(nch > 0)
            def _():
                lastp = jnp.bitwise_and(nch - 1, 1)
                pltpu.make_async_copy(
                    nmp_hbm.at[idxv.at[lastp]], rowsv.at[lastp], sem).wait()
                process(nch - 1, lastp)
            off = pl.multiple_of(b * (RB * ACC1W), 128)
            pltpu.sync_copy(accv.at[pl.ds(0, RB * ACC1W)],
                            out_hbm.at[pl.ds(off, RB * ACC1W)])
        return 0
    lax.fori_loop(0, 7, bucket, 0)


def _conv2_body(meta_hbm, nm2_hbm, st_hbm, neg_hbm,
                out_hbm,
                accv, metav, idxv, rowsv, stv, sem):
    wid = _wid()
    pltpu.sync_copy(st_hbm, stv.at[pl.ds(0, 512)])

    def bucket(kb, _):
        b = wid + kb * 32

        @pl.when(b < NBUCK)
        def _():
            sv = stv[pl.ds(b * 2, 16)]
            start = pl.multiple_of(sv[0], 128)
            cnt = sv[1]
            pltpu.sync_copy(neg_hbm, accv)
            nch = lax.div(cnt + 127, 128)

            def loadfire(kc, p):
                st = pl.multiple_of(start + kc * 128, 128)
                pltpu.sync_copy(meta_hbm.at[pl.ds(st, 128)],
                                metav.at[pl.ds(p * 144, 128)])

                def ib(j, _):
                    m = metav[pl.ds(p * 144 + j * 16, 16)]
                    srcv = jnp.bitwise_and(m, 0x1FFFF)
                    idxv[p, pl.ds(j * 16, 16)] = jnp.minimum(srcv, NS - 1)
                    return 0
                lax.fori_loop(0, 8, ib, 0, unroll=True)
                pltpu.async_copy(nm2_hbm.at[idxv.at[p]], rowsv.at[p], sem)

            def process(kc, p):
                rem = cnt - kc * 128

                def edge(i, _):
                    m = metav[pl.ds(p * 144 + i, 16)][0]
                    dl = lax.shift_right_logical(m, 18)
                    t = jnp.bitwise_and(lax.shift_right_logical(m, 17), 1)
                    dlv = jnp.where(i < rem, jnp.minimum(dl, TRASH), TRASH)
                    base = dlv * 64
                    for c in range(4):
                        v = rowsv[p, i, pl.ds(t * 64 + c * 16, 16)]
                        a = accv[pl.ds(base + c * 16, 16)]
                        accv[pl.ds(base + c * 16, 16)] = jnp.maximum(a, v)
                    return 0
                lax.fori_loop(0, 128, edge, 0, unroll=4)

            def chunk(kc, _):
                p = jnp.bitwise_and(kc, 1)
                loadfire(kc, p)

                @pl.when(kc > 0)
                def _():
                    pm = 1 - p
                    pltpu.make_async_copy(
                        nm2_hbm.at[idxv.at[pm]], rowsv.at[pm], sem).wait()
                    process(kc - 1, pm)
                return 0
            lax.fori_loop(0, nch, chunk, 0)

            @pl.when(nch > 0)
            def _():
                lastp = jnp.bitwise_and(nch - 1, 1)
                pltpu.make_async_copy(
                    nm2_hbm.at[idxv.at[lastp]], rowsv.at[lastp], sem).wait()
                process(nch - 1, lastp)
            off = pl.multiple_of(b * (RB * 64), 128)
            pltpu.sync_copy(accv.at[pl.ds(0, RB * 64)],
                            out_hbm.at[pl.ds(off, RB * 64)])
        return 0
    lax.fori_loop(0, 7, bucket, 0)


def kernel(x, edge_index, edge_attr, edge_type, Wn, bn, We, be, Wr1, Wer1,
           Wroot1, b1, Wr2, Wroot2, b2, Wc, bc):
    N = x.shape[0]
    E = edge_index.shape[1]
    f32 = jnp.float32

    src = edge_index[0]
    dst = edge_index[1]
    dst3 = dst.reshape(NGRID, 1, EB)
    src3 = src.reshape(NGRID, 1, EB)
    typ3 = edge_type.reshape(NGRID, 1, EB)

    hist3, rank3 = pl.pallas_call(
        _hr_body,
        grid=(NGRID,),
        in_specs=[pl.BlockSpec((1, 1, EB), lambda i: (i, 0, 0))],
        out_specs=[pl.BlockSpec((1, 25, NBP), lambda i: (i, 0, 0)),
                   pl.BlockSpec((1, 1, EB), lambda i: (i, 0, 0))],
        out_shape=[jax.ShapeDtypeStruct((NGRID, 25, NBP), f32),
                   jax.ShapeDtypeStruct((NGRID, 1, EB), f32)],
    )(dst3)

    hist = jnp.pad(hist3.reshape(3125, NBP), ((0, 75), (0, 0)))
    co, totI = pl.pallas_call(
        _scan_body,
        grid=(25,),
        in_specs=[pl.BlockSpec((128, NBP), lambda g: (g, 0))],
        out_specs=[pl.BlockSpec((128, NBP), lambda g: (g, 0)),
                   pl.BlockSpec((1, NBP), lambda g: (0, 0))],
        out_shape=[jax.ShapeDtypeStruct((3200, NBP), f32),
                   jax.ShapeDtypeStruct((1, NBP), f32)],
        scratch_shapes=[pltpu.VMEM((8, NBP), f32)],
    )(hist)
    startsI, cntsI = pl.pallas_call(
        _base_body,
        in_specs=[pl.BlockSpec((1, NBP), lambda: (0, 0))],
        out_specs=[pl.BlockSpec((1, 256), lambda: (0, 0)),
                   pl.BlockSpec((1, 256), lambda: (0, 0))],
        out_shape=[jax.ShapeDtypeStruct((1, 256), jnp.int32),
                   jax.ShapeDtypeStruct((1, 256), jnp.int32)],
    )(totI)
    co3 = co[:3125].reshape(NGRID, 25, NBP)

    pos3, meta3 = pl.pallas_call(
        _pos_body,
        grid=(NGRID,),
        in_specs=[pl.BlockSpec((1, 1, EB), lambda i: (i, 0, 0)),
                  pl.BlockSpec((1, 1, EB), lambda i: (i, 0, 0)),
                  pl.BlockSpec((1, 1, EB), lambda i: (i, 0, 0)),
                  pl.BlockSpec((1, 1, EB), lambda i: (i, 0, 0)),
                  pl.BlockSpec((1, 25, NBP), lambda i: (i, 0, 0)),
                  pl.BlockSpec((1, 256), lambda i: (0, 0))],
        out_specs=[pl.BlockSpec((1, 1, EB), lambda i: (i, 0, 0)),
                   pl.BlockSpec((1, 1, EB), lambda i: (i, 0, 0))],
        out_shape=[jax.ShapeDtypeStruct((NGRID, 1, EB), jnp.int32),
                   jax.ShapeDtypeStruct((NGRID, 1, EB), jnp.int32)],
    )(dst3, src3, typ3, rank3, co3, startsI)

    npad = MP2 - E
    pos_pad = jnp.concatenate(
        [pos3.reshape(E), MPOS0 + jnp.arange(npad, dtype=jnp.int32)])
    meta_pad = jnp.concatenate(
        [meta3.reshape(E),
         jnp.full((npad,), TRASH << 18, jnp.int32)])
    a0_pad = jnp.concatenate([edge_attr[:, 0], jnp.zeros((npad,), f32)])
    a1_pad = jnp.concatenate([edge_attr[:, 1], jnp.zeros((npad,), f32)])

    xp = jnp.pad(x, ((0, NS - N), (0, 8 - x.shape[1])))
    Wnp = jnp.pad(Wn, ((0, 8 - Wn.shape[0]), (0, 0)))
    n, nmp = pl.pallas_call(
        _tc1_body,
        grid=(98,),
        in_specs=[pl.BlockSpec((1024, 8), lambda i: (i, 0)),
                  pl.BlockSpec((8, 64), lambda i: (0, 0)),
                  pl.BlockSpec((1, 64), lambda i: (0, 0)),
                  pl.BlockSpec((2, 64, 64), lambda i: (0, 0, 0))],
        out_specs=[pl.BlockSpec((1024, 64), lambda i: (i, 0)),
                   pl.BlockSpec((1024, 128), lambda i: (i, 0))],
        out_shape=[jax.ShapeDtypeStruct((NS, 64), f32),
                   jax.ShapeDtypeStruct((NS, 128), f32)],
    )(xp, Wnp, bn.reshape(1, 64), Wr1)

    metaP, a0P, a1P = pl.kernel(
        _scatter_body,
        out_type=[jax.ShapeDtypeStruct((MP,), jnp.int32),
                  jax.ShapeDtypeStruct((MP,), f32),
                  jax.ShapeDtypeStruct((MP,), f32)],
        mesh=_mesh(),
        scratch_types=[pltpu.VMEM((2, 16, 128), jnp.int32),
                       pltpu.VMEM((2, 16, 128), jnp.int32),
                       pltpu.VMEM((2, 16, 128), f32),
                       pltpu.VMEM((2, 16, 128), f32),
                       pltpu.SemaphoreType.DMA,
                       pltpu.SemaphoreType.DMA,
                       pltpu.SemaphoreType.DMA,
                       pltpu.SemaphoreType.DMA,
                       pltpu.SemaphoreType.DMA,
                       pltpu.SemaphoreType.DMA],
    )(pos_pad.reshape(12800, 128), meta_pad.reshape(12800, 128),
      a0_pad.reshape(12800, 128), a1_pad.reshape(12800, 128))

    wpack = jnp.concatenate(
        [We[0], We[1], be,
         jnp.zeros((32,), f32).at[0].set(1.0)]).reshape(128)
    stpack = jnp.concatenate(
        [startsI.reshape(256, 1), cntsI.reshape(256, 1)], axis=1).reshape(512)
    acc1 = pl.kernel(
        _conv1_body,
        out_type=jax.ShapeDtypeStruct((NS * ACC1W,), f32),
        mesh=_mesh(),
        scratch_types=[pltpu.VMEM(((RB + 1) * ACC1W,), f32),
                       pltpu.VMEM((304,), jnp.int32),
                       pltpu.VMEM((304,), f32),
                       pltpu.VMEM((304,), f32),
                       pltpu.VMEM((2, 128), jnp.int32),
                       pltpu.VMEM((2, 128, 128), f32),
                       pltpu.VMEM((128,), f32),
                       pltpu.VMEM((528,), jnp.int32),
                       pltpu.SemaphoreType.DMA],
    )(metaP, a0P, a1P, nmp, wpack, stpack,
      jnp.zeros(((RB + 1) * ACC1W,), f32))

    h, nm2p = pl.pallas_call(
        _tc2_body_real,
        grid=(98,),
        in_specs=[pl.BlockSpec((1024, ACC1W), lambda i: (i, 0)),
                  pl.BlockSpec((1024, 64), lambda i: (i, 0)),
                  pl.BlockSpec((2, 32, 64), lambda i: (0, 0, 0)),
                  pl.BlockSpec((64, 64), lambda i: (0, 0)),
                  pl.BlockSpec((1, 64), lambda i: (0, 0)),
                  pl.BlockSpec((2, 64, 64), lambda i: (0, 0, 0))],
        out_specs=[pl.BlockSpec((1024, 64), lambda i: (i, 0)),
                   pl.BlockSpec((1024, 128), lambda i: (i, 0))],
        out_shape=[jax.ShapeDtypeStruct((NS, 64), f32),
                   jax.ShapeDtypeStruct((NS, 128), f32)],
    )(acc1.reshape(NS, ACC1W), n, Wer1, Wroot1, b1.reshape(1, 64), Wr2)

    agg2f = pl.kernel(
        _conv2_body,
        out_type=jax.ShapeDtypeStruct((NS * 64,), f32),
        mesh=_mesh(),
        scratch_types=[pltpu.VMEM(((RB + 1) * 64,), f32),
                       pltpu.VMEM((304,), jnp.int32),
                       pltpu.VMEM((2, 128), jnp.int32),
                       pltpu.VMEM((2, 128, 128), f32),
                       pltpu.VMEM((528,), jnp.int32),
                       pltpu.SemaphoreType.DMA],
    )(metaP, nm2p, stpack, jnp.full(((RB + 1) * 64,), NEG, f32))

    Wc128 = jnp.pad(Wc, ((0, 0), (0, 127)))
    bc128 = jnp.pad(bc.reshape(1, 1), ((0, 0), (0, 127)))
    out = pl.pallas_call(
        _tc3_body,
        grid=(98,),
        in_specs=[pl.BlockSpec((1024, 64), lambda i: (i, 0)),
                  pl.BlockSpec((1024, 64), lambda i: (i, 0)),
                  pl.BlockSpec((64, 64), lambda i: (0, 0)),
                  pl.BlockSpec((1, 64), lambda i: (0, 0)),
                  pl.BlockSpec((64, 128), lambda i: (0, 0)),
                  pl.BlockSpec((1, 128), lambda i: (0, 0))],
        out_specs=pl.BlockSpec((1024, 128), lambda i: (i, 0)),
        out_shape=jax.ShapeDtypeStruct((NS, 128), f32),
    )(agg2f.reshape(NS, 64), h, Wroot2, b2.reshape(1, 64), Wc128, bc128)
    return out[:N, :1]


def _tc2_body_real(acc_ref, n_ref, wer1_ref, wroot1_ref, b1_ref, wr2_ref,
                   h_ref, nm2_ref):
    acc = acc_ref[...]
    nm_s = acc[:, 0:64]
    e0 = acc[:, 64:96]
    c0 = acc[:, 96:97]
    e1 = acc[:, 112:144]
    c1 = acc[:, 144:145]
    deg = jnp.maximum(c0 + c1, 1.0)
    agg = (nm_s
           + jnp.dot(e0, wer1_ref[0], preferred_element_type=jnp.float32, precision=lax.Precision.HIGHEST)
           + jnp.dot(e1, wer1_ref[1], preferred_element_type=jnp.float32, precision=lax.Precision.HIGHEST)
           ) / deg
    hv = jax.nn.relu(
        agg + jnp.dot(n_ref[...], wroot1_ref[...],
                      preferred_element_type=jnp.float32,
                      precision=lax.Precision.HIGHEST) + b1_ref[...])
    h_ref[...] = hv
    nm2_ref[...] = jnp.concatenate(
        [jnp.dot(hv, wr2_ref[0], preferred_element_type=jnp.float32, precision=lax.Precision.HIGHEST),
         jnp.dot(hv, wr2_ref[1], preferred_element_type=jnp.float32, precision=lax.Precision.HIGHEST)], axis=1)
